# baseline (speedup 1.0000x reference)
import functools

import jax
import jax.numpy as jnp
from jax.experimental import pallas as pl

C = 128
P = 500
HEAD = 4
KNN = 40
B = 2
N = 2000
RBLK = 200

_NEG = -1e9


def _stage_call(stage_fn, tree, out_shapes):
    leaves, treedef = jax.tree.flatten(tree)
    n_in = len(leaves)

    def body(*refs):
        vals = [r[...] for r in refs[:n_in]]
        outs = stage_fn(jax.tree.unflatten(treedef, vals))
        if not isinstance(outs, (tuple, list)):
            outs = (outs,)
        for oref, o in zip(refs[n_in:], outs):
            oref[...] = o

    out_shape = [jax.ShapeDtypeStruct(s, jnp.float32) for s in out_shapes]
    res = pl.pallas_call(body, out_shape=out_shape)(*leaves)
    return res[0] if len(out_shapes) == 1 else res


_NT = (((1,), (0,)), ((), ()))


def _split(a):
    hi = a.astype(jnp.bfloat16)
    lo = (a - hi.astype(jnp.float32)).astype(jnp.bfloat16)
    return hi, lo


def _mm3(asp, bsp, dims):
    ah, al = asp
    bh, bl = bsp

    def dg(x, y):
        return jax.lax.dot_general(x, y, dims, preferred_element_type=jnp.float32)

    return dg(ah, bh) + (dg(ah, bl) + dg(al, bh))


def _dot3(a, b):
    return _mm3(_split(a), _split(b), _NT)


def _conv(p, x):
    wsp = _split(p["w"])
    return jnp.stack(
        [_mm3(wsp, _split(x[b]), _NT) for b in range(x.shape[0])]) + p["b"][None]


def _bn(x, p, eps=1e-5):
    m = x.mean(axis=(0, 2), keepdims=True)
    v = ((x - m) ** 2).mean(axis=(0, 2), keepdims=True)
    return (x - m) / jnp.sqrt(v + eps) * p["g"][None] + p["b"][None]


def _inorm(x, eps=1e-3):
    m = x.mean(axis=2, keepdims=True)
    v = ((x - m) ** 2).mean(axis=2, keepdims=True)
    return (x - m) / jnp.sqrt(v + eps)


def _relu(x):
    return jnp.maximum(x, 0.0)


def _pointca(p, x):
    w = _conv(p["ca_seed_conv"], _relu(_bn(_inorm(x), p["ca_seed_bn"])))
    w = jnp.tanh(_relu(w))
    w = w / jnp.maximum(jnp.sum(jnp.abs(w), axis=2, keepdims=True), 1e-12)
    x_sum = jnp.sum(x * w, axis=2, keepdims=True)
    out = _conv(p["ca_c2"], _relu(_bn(_conv(p["ca_c1"], x_sum), p["ca_bn"])))
    return jax.nn.sigmoid(out) * x


def _pointcn(p, x):
    out = _relu(_bn(_inorm(_conv(p["c1"], x)), p["bn1"]))
    out = _pointca(p, out)
    out = _relu(_bn(_inorm(_conv(p["c2"], out)), p["bn2"]))
    return out + x


def _frontend(tree):
    data, params = tree
    x = _conv(params["conv1"], data)
    for pp in params["pcn"]:
        x = _pointcn(pp, x)
    return x


def _kth_largest(pd, k):
    work = pd
    cur = None
    for _ in range(k):
        cur = jnp.max(work, axis=1, keepdims=True)
        work = jnp.where(work >= cur, -3e38, work)
    return cur


def _attention(tree, final):
    desc, p = tree
    hd = C // HEAD
    outs = []
    for b in range(B):
        db = desc[b]
        dsp = _split(db)
        q = _mm3(_split(p["q"]["w"]), dsp, _NT) + p["q"]["b"]
        k = _mm3(_split(p["k"]["w"]), dsp, _NT) + p["k"]["b"]
        v = _mm3(_split(p["v"]["w"]), dsp, _NT) + p["v"]["b"]
        qsp = _split(q)
        ksp = _split(k)
        vsp = _split(v)
        xx = jnp.sum(db * db, axis=0, keepdims=True)
        _TT = (((0,), (0,)), ((), ()))

        def pd_block(r):
            rsp = tuple(h[:, r * RBLK:(r + 1) * RBLK] for h in dsp)
            g = _mm3(rsp, dsp, _TT)
            xxr = xx[:, r * RBLK:(r + 1) * RBLK]
            return 2.0 * g - (jnp.transpose(xxr) + xx)

        t = jnp.concatenate(
            [_kth_largest(pd_block(r), KNN) for r in range(N // RBLK)], axis=0)
        t_row = jnp.transpose(t)

        av_blocks = []
        for r in range(N // RBLK):
            pd = pd_block(r)
            tr = t[r * RBLK:(r + 1) * RBLK]
            mask = jnp.logical_and(pd >= tr, pd >= t_row)
            head_outs = []
            for h in range(HEAD):
                qhsp = tuple(z[h * hd:(h + 1) * hd, r * RBLK:(r + 1) * RBLK] for z in qsp)
                khsp = tuple(z[h * hd:(h + 1) * hd] for z in ksp)
                vhsp = tuple(z[h * hd:(h + 1) * hd] for z in vsp)
                s = _mm3(qhsp, khsp, (((0,), (0,)), ((), ())))
                s = s * (1.0 / (hd ** 0.5))
                s = jnp.where(mask, s, _NEG)
                s = s - jnp.max(s, axis=1, keepdims=True)
                e = jnp.exp(s)
                pr = e / jnp.sum(e, axis=1, keepdims=True)
                o = _mm3(vhsp, _split(pr), (((1,), (1,)), ((), ())))
                head_outs.append(o)
            av_blocks.append(jnp.concatenate(head_outs, axis=0))
        av = jnp.concatenate(av_blocks, axis=1)
        av = _dot3(p["mh"]["w"], av) + p["mh"]["b"]
        cat = jnp.concatenate([db, av], axis=0)
        c1 = _dot3(p["cat1"]["w"], cat) + p["cat1"]["b"]
        outs.append((db, c1))

    c1s = jnp.stack([o[1] for o in outs])
    c1s = _relu(_bn(c1s, p["cat_bn"]))
    res = []
    for b in range(B):
        c2 = _dot3(p["cat2"]["w"], c1s[b]) + p["cat2"]["b"]
        y = outs[b][0] + c2
        if final:
            y = _dot3(p["out"]["w"], y) + p["out"]["b"]
            res.append(y)
        else:
            res.append(y[None])
    return jnp.concatenate(res, axis=0)


def _oafilter(p, x):
    out = _conv(p["c1"], _relu(_bn(_inorm(x), p["bn1"])))
    out = jnp.transpose(out, (0, 2, 1))
    out = out + _conv(p["c2"], _relu(_bn(out, p["bn2"])))
    out = jnp.transpose(out, (0, 2, 1))
    out = _conv(p["c3"], _relu(_bn(_inorm(out), p["bn3"])))
    return out + x


def _middle(tree):
    x, params = tree
    embed = _conv(params["down"]["conv"], _relu(_bn(_inorm(x), params["down"]["bn"])))
    embed = embed - jnp.max(embed, axis=2, keepdims=True)
    e = jnp.exp(embed)
    s = e / jnp.sum(e, axis=2, keepdims=True)
    x2 = jnp.stack([
        _mm3(_split(x[b]), _split(s[b]), (((1,), (1,)), ((), ())))
        for b in range(B)])
    for pp in params["oaf"]:
        x2 = _oafilter(pp, x2)
    embed = _conv(params["up"]["conv"], _relu(_bn(_inorm(x), params["up"]["bn"])))
    embed = embed - jnp.max(embed, axis=1, keepdims=True)
    e = jnp.exp(embed)
    s = e / jnp.sum(e, axis=1, keepdims=True)
    x_up = jnp.stack([
        _dot3(x2[b], s[b])
        for b in range(B)])
    cat = jnp.concatenate([x, x_up], axis=1)
    out = _relu(_bn(_conv(params["l12_conv"], cat), params["l12_bn"]))
    return out


def _col(v):
    return v.reshape(-1, 1)


def _prep(tree):
    return jax.tree.map(lambda a: _col(a) if a.ndim == 1 else a, tree)


def kernel(data, params):
    data = data[..., 0]
    params = _prep(params)

    x = _stage_call(_frontend, (data, {"conv1": params["conv1"], "pcn": params["pcn"]}),
                    [(B, C, N)])
    attn1 = dict(params["attn1"])
    x = _stage_call(functools.partial(_attention, final=False), (x, attn1), [(B, C, N)])
    mid = {k: params[k] for k in ("down", "oaf", "up", "l12_conv", "l12_bn")}
    x = _stage_call(_middle, (x, mid), [(B, C, N)])
    attn2 = dict(params["attn2"])
    attn2["out"] = params["out"]
    logits = _stage_call(functools.partial(_attention, final=True), (x, attn2), [(B, N)])
    return logits


# speedup vs baseline: 1.3273x; 1.3273x over previous
import functools

import jax
import jax.numpy as jnp
from jax.experimental import pallas as pl

C = 128
P = 500
HEAD = 4
KNN = 40
B = 2
N = 2000
RBLK = 200

_NEG = -1e9


def _stage_call(stage_fn, tree, out_shapes):
    leaves, treedef = jax.tree.flatten(tree)
    n_in = len(leaves)

    def body(*refs):
        vals = [r[...] for r in refs[:n_in]]
        outs = stage_fn(jax.tree.unflatten(treedef, vals))
        if not isinstance(outs, (tuple, list)):
            outs = (outs,)
        for oref, o in zip(refs[n_in:], outs):
            oref[...] = o

    out_shape = [jax.ShapeDtypeStruct(s, jnp.float32) for s in out_shapes]
    res = pl.pallas_call(body, out_shape=out_shape)(*leaves)
    return res[0] if len(out_shapes) == 1 else res


_NT = (((1,), (0,)), ((), ()))


def _split(a):
    return a


def _mm3(a, b, dims):
    return jax.lax.dot_general(a, b, dims, preferred_element_type=jnp.float32)


def _dot3(a, b):
    return _mm3(a, b, _NT)


def _conv(p, x):
    wsp = _split(p["w"])
    return jnp.stack(
        [_mm3(wsp, _split(x[b]), _NT) for b in range(x.shape[0])]) + p["b"][None]


def _bn(x, p, eps=1e-5):
    m = x.mean(axis=(0, 2), keepdims=True)
    v = ((x - m) ** 2).mean(axis=(0, 2), keepdims=True)
    return (x - m) / jnp.sqrt(v + eps) * p["g"][None] + p["b"][None]


def _inorm(x, eps=1e-3):
    m = x.mean(axis=2, keepdims=True)
    v = ((x - m) ** 2).mean(axis=2, keepdims=True)
    return (x - m) / jnp.sqrt(v + eps)


def _relu(x):
    return jnp.maximum(x, 0.0)


def _pointca(p, x):
    w = _conv(p["ca_seed_conv"], _relu(_bn(_inorm(x), p["ca_seed_bn"])))
    w = jnp.tanh(_relu(w))
    w = w / jnp.maximum(jnp.sum(jnp.abs(w), axis=2, keepdims=True), 1e-12)
    x_sum = jnp.sum(x * w, axis=2, keepdims=True)
    out = _conv(p["ca_c2"], _relu(_bn(_conv(p["ca_c1"], x_sum), p["ca_bn"])))
    return jax.nn.sigmoid(out) * x


def _pointcn(p, x):
    out = _relu(_bn(_inorm(_conv(p["c1"], x)), p["bn1"]))
    out = _pointca(p, out)
    out = _relu(_bn(_inorm(_conv(p["c2"], out)), p["bn2"]))
    return out + x


def _frontend(tree):
    data, params = tree
    x = _conv(params["conv1"], data)
    for pp in params["pcn"]:
        x = _pointcn(pp, x)
    return x


def _kth_largest(pd, k):
    work = pd
    cur = None
    for _ in range(k):
        cur = jnp.max(work, axis=1, keepdims=True)
        work = jnp.where(work >= cur, -3e38, work)
    return cur


def _attention(tree, final):
    desc, p = tree
    hd = C // HEAD
    outs = []
    for b in range(B):
        db = desc[b]
        dsp = _split(db)
        q = _mm3(_split(p["q"]["w"]), dsp, _NT) + p["q"]["b"]
        k = _mm3(_split(p["k"]["w"]), dsp, _NT) + p["k"]["b"]
        v = _mm3(_split(p["v"]["w"]), dsp, _NT) + p["v"]["b"]
        qsp = _split(q)
        ksp = _split(k)
        vsp = _split(v)
        xx = jnp.sum(db * db, axis=0, keepdims=True)
        _TT = (((0,), (0,)), ((), ()))

        def pd_block(r):
            rsp = dsp[:, r * RBLK:(r + 1) * RBLK]
            g = _mm3(rsp, dsp, _TT)
            xxr = xx[:, r * RBLK:(r + 1) * RBLK]
            return 2.0 * g - (jnp.transpose(xxr) + xx)

        pds = [pd_block(r) for r in range(N // RBLK)]
        t = jnp.concatenate(
            [_kth_largest(pds[r], KNN) for r in range(N // RBLK)], axis=0)
        t_row = jnp.transpose(t)

        av_blocks = []
        for r in range(N // RBLK):
            pd = pds[r]
            tr = t[r * RBLK:(r + 1) * RBLK]
            mask = jnp.logical_and(pd >= tr, pd >= t_row)
            head_outs = []
            for h in range(HEAD):
                qhsp = qsp[h * hd:(h + 1) * hd, r * RBLK:(r + 1) * RBLK]
                khsp = ksp[h * hd:(h + 1) * hd]
                vhsp = vsp[h * hd:(h + 1) * hd]
                s = _mm3(qhsp, khsp, (((0,), (0,)), ((), ())))
                s = s * (1.0 / (hd ** 0.5))
                s = jnp.where(mask, s, _NEG)
                s = s - jnp.max(s, axis=1, keepdims=True)
                e = jnp.exp(s)
                pr = e / jnp.sum(e, axis=1, keepdims=True)
                o = _mm3(vhsp, _split(pr), (((1,), (1,)), ((), ())))
                head_outs.append(o)
            av_blocks.append(jnp.concatenate(head_outs, axis=0))
        av = jnp.concatenate(av_blocks, axis=1)
        av = _dot3(p["mh"]["w"], av) + p["mh"]["b"]
        cat = jnp.concatenate([db, av], axis=0)
        c1 = _dot3(p["cat1"]["w"], cat) + p["cat1"]["b"]
        outs.append((db, c1))

    c1s = jnp.stack([o[1] for o in outs])
    c1s = _relu(_bn(c1s, p["cat_bn"]))
    res = []
    for b in range(B):
        c2 = _dot3(p["cat2"]["w"], c1s[b]) + p["cat2"]["b"]
        y = outs[b][0] + c2
        if final:
            y = _dot3(p["out"]["w"], y) + p["out"]["b"]
            res.append(y)
        else:
            res.append(y[None])
    return jnp.concatenate(res, axis=0)


def _oafilter(p, x):
    out = _conv(p["c1"], _relu(_bn(_inorm(x), p["bn1"])))
    out = jnp.transpose(out, (0, 2, 1))
    out = out + _conv(p["c2"], _relu(_bn(out, p["bn2"])))
    out = jnp.transpose(out, (0, 2, 1))
    out = _conv(p["c3"], _relu(_bn(_inorm(out), p["bn3"])))
    return out + x


def _middle(tree):
    x, params = tree
    embed = _conv(params["down"]["conv"], _relu(_bn(_inorm(x), params["down"]["bn"])))
    embed = embed - jnp.max(embed, axis=2, keepdims=True)
    e = jnp.exp(embed)
    s = e / jnp.sum(e, axis=2, keepdims=True)
    x2 = jnp.stack([
        _mm3(_split(x[b]), _split(s[b]), (((1,), (1,)), ((), ())))
        for b in range(B)])
    for pp in params["oaf"]:
        x2 = _oafilter(pp, x2)
    embed = _conv(params["up"]["conv"], _relu(_bn(_inorm(x), params["up"]["bn"])))
    embed = embed - jnp.max(embed, axis=1, keepdims=True)
    e = jnp.exp(embed)
    s = e / jnp.sum(e, axis=1, keepdims=True)
    x_up = jnp.stack([
        _dot3(x2[b], s[b])
        for b in range(B)])
    cat = jnp.concatenate([x, x_up], axis=1)
    out = _relu(_bn(_conv(params["l12_conv"], cat), params["l12_bn"]))
    return out


def _col(v):
    return v.reshape(-1, 1)


def _prep(tree):
    return jax.tree.map(lambda a: _col(a) if a.ndim == 1 else a, tree)


def kernel(data, params):
    data = data[..., 0]
    params = _prep(params)

    x = _stage_call(_frontend, (data, {"conv1": params["conv1"], "pcn": params["pcn"]}),
                    [(B, C, N)])
    attn1 = dict(params["attn1"])
    x = _stage_call(functools.partial(_attention, final=False), (x, attn1), [(B, C, N)])
    mid = {k: params[k] for k in ("down", "oaf", "up", "l12_conv", "l12_bn")}
    x = _stage_call(_middle, (x, mid), [(B, C, N)])
    attn2 = dict(params["attn2"])
    attn2["out"] = params["out"]
    logits = _stage_call(functools.partial(_attention, final=True), (x, attn2), [(B, N)])
    return logits


# speedup vs baseline: 1.3933x; 1.0497x over previous
import functools

import jax
import jax.numpy as jnp
from jax.experimental import pallas as pl

C = 128
P = 500
HEAD = 4
KNN = 40
B = 2
N = 2000
RBLK = 200

_NEG = -1e9


def _stage_call(stage_fn, tree, out_shapes):
    leaves, treedef = jax.tree.flatten(tree)
    n_in = len(leaves)

    def body(*refs):
        vals = [r[...] for r in refs[:n_in]]
        outs = stage_fn(jax.tree.unflatten(treedef, vals))
        if not isinstance(outs, (tuple, list)):
            outs = (outs,)
        for oref, o in zip(refs[n_in:], outs):
            oref[...] = o

    out_shape = [jax.ShapeDtypeStruct(s, jnp.float32) for s in out_shapes]
    res = pl.pallas_call(body, out_shape=out_shape)(*leaves)
    return res[0] if len(out_shapes) == 1 else res


_NT = (((1,), (0,)), ((), ()))


def _split(a):
    return a


def _mm3(a, b, dims):
    return jax.lax.dot_general(a, b, dims, preferred_element_type=jnp.float32)


def _dot3(a, b):
    return _mm3(a, b, _NT)


def _conv(p, x):
    wsp = _split(p["w"])
    return jnp.stack(
        [_mm3(wsp, _split(x[b]), _NT) for b in range(x.shape[0])]) + p["b"][None]


def _bn(x, p, eps=1e-5):
    m = x.mean(axis=(0, 2), keepdims=True)
    v = ((x - m) ** 2).mean(axis=(0, 2), keepdims=True)
    return (x - m) / jnp.sqrt(v + eps) * p["g"][None] + p["b"][None]


def _inorm(x, eps=1e-3):
    m = x.mean(axis=2, keepdims=True)
    v = ((x - m) ** 2).mean(axis=2, keepdims=True)
    return (x - m) / jnp.sqrt(v + eps)


def _relu(x):
    return jnp.maximum(x, 0.0)


def _pointca(p, x):
    w = _conv(p["ca_seed_conv"], _relu(_bn(_inorm(x), p["ca_seed_bn"])))
    w = jnp.tanh(_relu(w))
    w = w / jnp.maximum(jnp.sum(jnp.abs(w), axis=2, keepdims=True), 1e-12)
    x_sum = jnp.sum(x * w, axis=2, keepdims=True)
    out = _conv(p["ca_c2"], _relu(_bn(_conv(p["ca_c1"], x_sum), p["ca_bn"])))
    return jax.nn.sigmoid(out) * x


def _pointcn(p, x):
    out = _relu(_bn(_inorm(_conv(p["c1"], x)), p["bn1"]))
    out = _pointca(p, out)
    out = _relu(_bn(_inorm(_conv(p["c2"], out)), p["bn2"]))
    return out + x


def _frontend(tree):
    data, params = tree
    x = _conv(params["conv1"], data)
    for pp in params["pcn"]:
        x = _pointcn(pp, x)
    return x


def _kth_largest(pd, k):
    work = pd
    cur = None
    for _ in range(k):
        cur = jnp.max(work, axis=1, keepdims=True)
        work = jnp.where(work >= cur, -3e38, work)
    return cur


def _attention(tree, final):
    desc, p = tree
    hd = C // HEAD
    outs = []
    for b in range(B):
        db = desc[b]
        dsp = _split(db)
        q = _mm3(_split(p["q"]["w"]), dsp, _NT) + p["q"]["b"]
        k = _mm3(_split(p["k"]["w"]), dsp, _NT) + p["k"]["b"]
        v = _mm3(_split(p["v"]["w"]), dsp, _NT) + p["v"]["b"]
        qsp = _split(q)
        ksp = _split(k)
        vsp = _split(v)
        xx = jnp.sum(db * db, axis=0, keepdims=True)
        _TT = (((0,), (0,)), ((), ()))

        def pd_block(r):
            rsp = dsp[:, r * RBLK:(r + 1) * RBLK]
            g = _mm3(rsp, dsp, _TT)
            xxr = xx[:, r * RBLK:(r + 1) * RBLK]
            return 2.0 * g - (jnp.transpose(xxr) + xx)

        t = jnp.concatenate(
            [_kth_largest(pd_block(r), KNN) for r in range(N // RBLK)], axis=0)
        t_row = jnp.transpose(t)

        av_blocks = []
        for r in range(N // RBLK):
            pd = pd_block(r)
            tr = t[r * RBLK:(r + 1) * RBLK]
            mask = jnp.logical_and(pd >= tr, pd >= t_row)
            logmf = jnp.where(mask, 0.0, -3e38)
            head_outs = []
            for h in range(HEAD):
                qhsp = qsp[h * hd:(h + 1) * hd, r * RBLK:(r + 1) * RBLK]
                khsp = ksp[h * hd:(h + 1) * hd]
                vhsp = vsp[h * hd:(h + 1) * hd]
                s = _mm3(qhsp, khsp, (((0,), (0,)), ((), ())))
                s = s * (1.0 / (hd ** 0.5)) + logmf
                e = jnp.exp(s - jnp.max(s, axis=1, keepdims=True))
                rs = 1.0 / jnp.sum(e, axis=1, keepdims=True)
                o = _mm3(vhsp, _split(e), (((1,), (1,)), ((), ())))
                o = o * jnp.transpose(rs)
                head_outs.append(o)
            av_blocks.append(jnp.concatenate(head_outs, axis=0))
        av = jnp.concatenate(av_blocks, axis=1)
        av = _dot3(p["mh"]["w"], av) + p["mh"]["b"]
        cat = jnp.concatenate([db, av], axis=0)
        c1 = _dot3(p["cat1"]["w"], cat) + p["cat1"]["b"]
        outs.append((db, c1))

    c1s = jnp.stack([o[1] for o in outs])
    c1s = _relu(_bn(c1s, p["cat_bn"]))
    res = []
    for b in range(B):
        c2 = _dot3(p["cat2"]["w"], c1s[b]) + p["cat2"]["b"]
        y = outs[b][0] + c2
        if final:
            y = _dot3(p["out"]["w"], y) + p["out"]["b"]
            res.append(y)
        else:
            res.append(y[None])
    return jnp.concatenate(res, axis=0)


def _oafilter(p, x):
    out = _conv(p["c1"], _relu(_bn(_inorm(x), p["bn1"])))
    out = jnp.transpose(out, (0, 2, 1))
    out = out + _conv(p["c2"], _relu(_bn(out, p["bn2"])))
    out = jnp.transpose(out, (0, 2, 1))
    out = _conv(p["c3"], _relu(_bn(_inorm(out), p["bn3"])))
    return out + x


def _middle(tree):
    x, params = tree
    embed = _conv(params["down"]["conv"], _relu(_bn(_inorm(x), params["down"]["bn"])))
    embed = embed - jnp.max(embed, axis=2, keepdims=True)
    e = jnp.exp(embed)
    s = e / jnp.sum(e, axis=2, keepdims=True)
    x2 = jnp.stack([
        _mm3(_split(x[b]), _split(s[b]), (((1,), (1,)), ((), ())))
        for b in range(B)])
    for pp in params["oaf"]:
        x2 = _oafilter(pp, x2)
    embed = _conv(params["up"]["conv"], _relu(_bn(_inorm(x), params["up"]["bn"])))
    embed = embed - jnp.max(embed, axis=1, keepdims=True)
    e = jnp.exp(embed)
    s = e / jnp.sum(e, axis=1, keepdims=True)
    x_up = jnp.stack([
        _dot3(x2[b], s[b])
        for b in range(B)])
    cat = jnp.concatenate([x, x_up], axis=1)
    out = _relu(_bn(_conv(params["l12_conv"], cat), params["l12_bn"]))
    return out


def _col(v):
    return v.reshape(-1, 1)


def _prep(tree):
    return jax.tree.map(lambda a: _col(a) if a.ndim == 1 else a, tree)


def kernel(data, params):
    data = data[..., 0]
    params = _prep(params)

    x = _stage_call(_frontend, (data, {"conv1": params["conv1"], "pcn": params["pcn"]}),
                    [(B, C, N)])
    attn1 = dict(params["attn1"])
    x = _stage_call(functools.partial(_attention, final=False), (x, attn1), [(B, C, N)])
    mid = {k: params[k] for k in ("down", "oaf", "up", "l12_conv", "l12_bn")}
    x = _stage_call(_middle, (x, mid), [(B, C, N)])
    attn2 = dict(params["attn2"])
    attn2["out"] = params["out"]
    logits = _stage_call(functools.partial(_attention, final=True), (x, attn2), [(B, N)])
    return logits
